# Initial kernel scaffold; baseline (speedup 1.0000x reference)
import functools

import jax
import jax.numpy as jnp
from jax import lax
from jax.experimental import pallas as pl
from jax.experimental.pallas import tpu as pltpu
from jax.experimental.pallas import tpu_sc as plsc

f32 = jnp.float32

_NC = 2
_NS = 16
_NW = _NC * _NS
_CH = 80

_EBLK = 2500
_NBLK = 2500


def _relu(v):
    return jnp.maximum(v, 0.0)


def _node_proj_body(x_ref, wa_ref, wb_ref, b_ref, a_ref, b_out_ref):
    xv = x_ref[...]
    a_ref[...] = jnp.dot(xv, wa_ref[...], preferred_element_type=f32)
    b_out_ref[...] = jnp.dot(xv, wb_ref[...], preferred_element_type=f32) + b_ref[...]


def _tc_node_proj(x, wa, wb, b1):
    n, d = x.shape
    h = wa.shape[1]
    blk = _NBLK
    return pl.pallas_call(
        _node_proj_body,
        grid=(n // blk,),
        in_specs=[
            pl.BlockSpec((blk, d), lambda i: (i, 0)),
            pl.BlockSpec((d, h), lambda i: (0, 0)),
            pl.BlockSpec((d, h), lambda i: (0, 0)),
            pl.BlockSpec((1, h), lambda i: (0, 0)),
        ],
        out_specs=[
            pl.BlockSpec((blk, h), lambda i: (i, 0)),
            pl.BlockSpec((blk, h), lambda i: (i, 0)),
        ],
        out_shape=[
            jax.ShapeDtypeStruct((n, h), f32),
            jax.ShapeDtypeStruct((n, h), f32),
        ],
    )(x, wa, wb, b1)


def _edge0_body(gs_ref, gd_ref, ea_ref, wc_ref, w2_ref, b2_ref, out_ref):
    pre = gs_ref[...] + gd_ref[...] + jnp.dot(
        ea_ref[...], wc_ref[...], preferred_element_type=f32)
    hv = _relu(pre)
    out_ref[...] = jnp.dot(hv, w2_ref[...], preferred_element_type=f32) + b2_ref[...]


def _tc_edge0(gs, gd, ea, wc, w2, b2):
    e, h = gs.shape
    de = ea.shape[1]
    blk = _EBLK
    return pl.pallas_call(
        _edge0_body,
        grid=(e // blk,),
        in_specs=[
            pl.BlockSpec((blk, h), lambda i: (i, 0)),
            pl.BlockSpec((blk, h), lambda i: (i, 0)),
            pl.BlockSpec((blk, de), lambda i: (i, 0)),
            pl.BlockSpec((de, h), lambda i: (0, 0)),
            pl.BlockSpec((h, h), lambda i: (0, 0)),
            pl.BlockSpec((1, h), lambda i: (0, 0)),
        ],
        out_specs=pl.BlockSpec((blk, h), lambda i: (i, 0)),
        out_shape=jax.ShapeDtypeStruct((e, h), f32),
    )(gs, gd, ea, wc, w2, b2)


def _node_body(x_ref, sd_ref, cnt_ref, ss_ref, wx_ref, wp_ref, wf_ref, b1_ref,
               w2_ref, b2_ref, wa_ref, wb_ref, eb1_ref, a_out, b_out):
    inv = 1.0 / jnp.maximum(cnt_ref[..., 0:1], 1.0)
    past = sd_ref[...] * inv
    xv = x_ref[...]
    hv = _relu(
        jnp.dot(xv, wx_ref[...], preferred_element_type=f32)
        + jnp.dot(past, wp_ref[...], preferred_element_type=f32)
        + jnp.dot(ss_ref[...], wf_ref[...], preferred_element_type=f32)
        + b1_ref[...])
    x0 = jnp.dot(hv, w2_ref[...], preferred_element_type=f32) + b2_ref[...] + xv
    a_out[...] = jnp.dot(x0, wa_ref[...], preferred_element_type=f32)
    b_out[...] = jnp.dot(x0, wb_ref[...], preferred_element_type=f32) + eb1_ref[...]


def _tc_node(x, sd, cnt, ss, wx, wp, wf, b1, w2, b2, wa, wb, eb1):
    n, d = x.shape
    h = w2.shape[1]
    blk = _NBLK
    wspec = pl.BlockSpec((d, h), lambda i: (0, 0))
    bspec = pl.BlockSpec((1, h), lambda i: (0, 0))
    return pl.pallas_call(
        _node_body,
        grid=(n // blk,),
        in_specs=[
            pl.BlockSpec((blk, d), lambda i: (i, 0)),
            pl.BlockSpec((blk, h), lambda i: (i, 0)),
            pl.BlockSpec((blk, cnt.shape[1]), lambda i: (i, 0)),
            pl.BlockSpec((blk, h), lambda i: (i, 0)),
            wspec, wspec, wspec, bspec, wspec, bspec, wspec, wspec, bspec,
        ],
        out_specs=[
            pl.BlockSpec((blk, h), lambda i: (i, 0)),
            pl.BlockSpec((blk, h), lambda i: (i, 0)),
        ],
        out_shape=[
            jax.ShapeDtypeStruct((n, h), f32),
            jax.ShapeDtypeStruct((n, h), f32),
        ],
    )(x, sd, cnt, ss, wx, wp, wf, b1, w2, b2, wa, wb, eb1)


def _edge1_body(gs_ref, gd_ref, ea_ref, wc_ref, w2_ref, b2_ref,
                pw1_ref, pb1_ref, pw2_ref, pb2_ref, out_ref):
    ea = ea_ref[...]
    hv = _relu(gs_ref[...] + gd_ref[...] + jnp.dot(
        ea, wc_ref[...], preferred_element_type=f32))
    ea1 = jnp.dot(hv, w2_ref[...], preferred_element_type=f32) + b2_ref[...] + ea
    ph = _relu(jnp.dot(ea1, pw1_ref[...], preferred_element_type=f32) + pb1_ref[...])
    out_ref[...] = jnp.sum(ph * pw2_ref[...], axis=1, keepdims=True) + pb2_ref[...]


def _tc_edge1(gs, gd, ea, wc, w2, b2, pw1, pb1, pw2row, pb2):
    e, h = gs.shape
    blk = _EBLK
    wspec = pl.BlockSpec((h, h), lambda i: (0, 0))
    bspec = pl.BlockSpec((1, h), lambda i: (0, 0))
    return pl.pallas_call(
        _edge1_body,
        grid=(e // blk,),
        in_specs=[
            pl.BlockSpec((blk, h), lambda i: (i, 0)),
            pl.BlockSpec((blk, h), lambda i: (i, 0)),
            pl.BlockSpec((blk, h), lambda i: (i, 0)),
            wspec, wspec, bspec, wspec, bspec, bspec,
            pl.BlockSpec((1, 1), lambda i: (0, 0)),
        ],
        out_specs=pl.BlockSpec((blk, 1), lambda i: (i, 0)),
        out_shape=jax.ShapeDtypeStruct((e, 1), f32),
    )(gs, gd, ea, wc, w2, b2, pw1, pb1, pw2row, pb2)


_MESH = plsc.VectorSubcoreMesh(
    core_axis_name="c", subcore_axis_name="s", num_cores=_NC, num_subcores=_NS)


def _sc_gather_body(ta, tb, srci, dsti, out_s, out_d,
                    sidx, didx, rows_a, rows_b, sem_a, sem_b):
    ew = srci.shape[0] // _NW
    nch = ew // _CH
    wid = lax.axis_index("s") * _NC + lax.axis_index("c")

    def chunk(i, carry):
        base = pl.multiple_of(wid * ew + i * _CH, _CH)
        pltpu.sync_copy(srci.at[pl.ds(base, _CH)], sidx)
        pltpu.sync_copy(dsti.at[pl.ds(base, _CH)], didx)
        ca = pltpu.async_copy(ta.at[sidx], rows_a, sem_a)
        cb = pltpu.async_copy(tb.at[didx], rows_b, sem_b)
        ca.wait()
        cb.wait()
        pltpu.sync_copy(rows_a, out_s.at[pl.ds(base, _CH)])
        pltpu.sync_copy(rows_b, out_d.at[pl.ds(base, _CH)])
        return carry

    lax.fori_loop(0, nch, chunk, 0)


def _sc_gather2(ta, tb, srci, dsti):
    e = srci.shape[0]
    d = ta.shape[1]
    return pl.kernel(
        _sc_gather_body,
        out_type=(
            jax.ShapeDtypeStruct((e, d), f32),
            jax.ShapeDtypeStruct((e, d), f32),
        ),
        mesh=_MESH,
        scratch_types=[
            pltpu.VMEM((_CH,), jnp.int32),
            pltpu.VMEM((_CH,), jnp.int32),
            pltpu.VMEM((_CH, d), f32),
            pltpu.VMEM((_CH, d), f32),
            pltpu.SemaphoreType.DMA,
            pltpu.SemaphoreType.DMA,
        ],
    )(ta, tb, srci, dsti)


def _sc_scatter_body(ea, dsti, srci, z128, z16, ones_h,
                     sd_out, cnt_out, ss_out,
                     idx_v, rows_v, ones_v, acc, cnt_acc):
    n = z128.shape[0]
    e = dsti.shape[0]
    rpt = n // _NS
    ept = e // _NS
    nch = ept // _CH
    c = lax.axis_index("c")
    s = lax.axis_index("s")
    r0 = pl.multiple_of(s * rpt, 8)

    pltpu.sync_copy(z128.at[pl.ds(r0, rpt)], acc.at[pl.ds(r0, rpt)])
    pltpu.sync_copy(z16.at[pl.ds(r0, rpt)], cnt_acc.at[pl.ds(r0, rpt)])
    pltpu.sync_copy(ones_h, ones_v)
    plsc.subcore_barrier()

    def chunk(i, carry):
        base = pl.multiple_of(s * ept + i * _CH, _CH)

        @pl.when(c == 0)
        def _():
            pltpu.sync_copy(dsti.at[pl.ds(base, _CH)], idx_v)

        @pl.when(c == 1)
        def _():
            pltpu.sync_copy(srci.at[pl.ds(base, _CH)], idx_v)

        pltpu.sync_copy(ea.at[pl.ds(base, _CH)], rows_v)
        pltpu.sync_copy(rows_v, acc.at[idx_v], add=True)

        @pl.when(c == 0)
        def _():
            pltpu.sync_copy(ones_v, cnt_acc.at[idx_v], add=True)

        return carry

    lax.fori_loop(0, nch, chunk, 0)
    plsc.subcore_barrier()

    @pl.when(c == 0)
    def _():
        pltpu.sync_copy(acc.at[pl.ds(r0, rpt)], sd_out.at[pl.ds(r0, rpt)])
        pltpu.sync_copy(cnt_acc.at[pl.ds(r0, rpt)], cnt_out.at[pl.ds(r0, rpt)])

    @pl.when(c == 1)
    def _():
        pltpu.sync_copy(acc.at[pl.ds(r0, rpt)], ss_out.at[pl.ds(r0, rpt)])


def _sc_scatter(ea, dsti, srci, z128, z16, ones_h):
    n = z128.shape[0]
    h = ea.shape[1]
    return pl.kernel(
        _sc_scatter_body,
        out_type=(
            jax.ShapeDtypeStruct((n, h), f32),
            jax.ShapeDtypeStruct((n, 16), f32),
            jax.ShapeDtypeStruct((n, h), f32),
        ),
        mesh=_MESH,
        scratch_types=[
            pltpu.VMEM((_CH,), jnp.int32),
            pltpu.VMEM((_CH, h), f32),
            pltpu.VMEM((_CH, 16), f32),
            pltpu.VMEM_SHARED((n, h), f32),
            pltpu.VMEM_SHARED((n, 16), f32),
        ],
    )(ea, dsti, srci, z128, z16, ones_h)


def kernel(x, edge_index, edge_attr,
           e0_W1, e0_b1, e0_W2, e0_b2, n0_W1, n0_b1, n0_W2, n0_b2,
           e1_W1, e1_b1, e1_W2, e1_b2, n1_W1, n1_b1, n1_W2, n1_b2,
           p_W1, p_b1, p_W2, p_b2):
    n, d = x.shape
    h = e0_W2.shape[1]
    src = edge_index[0]
    dst = edge_index[1]

    xa, xb = _tc_node_proj(x, e0_W1[:d], e0_W1[d:2 * d], e0_b1[None, :])
    gs, gd = _sc_gather2(xa, xb, src, dst)
    ea0 = _tc_edge0(gs, gd, edge_attr, e0_W1[2 * d:], e0_W2, e0_b2[None, :])

    z128 = jnp.zeros((n, h), f32)
    z16 = jnp.zeros((n, 16), f32)
    ones_h = jnp.ones((_CH, 16), f32)
    sd, cnt, ss = _sc_scatter(ea0, dst, src, z128, z16, ones_h)

    a1, b1v = _tc_node(
        x, sd, cnt, ss,
        n0_W1[:d], n0_W1[d:d + h], n0_W1[d + h:], n0_b1[None, :],
        n0_W2, n0_b2[None, :],
        e1_W1[:h], e1_W1[h:2 * h], e1_b1[None, :])

    g1s, g1d = _sc_gather2(a1, b1v, src, dst)

    out = _tc_edge1(
        g1s, g1d, ea0, e1_W1[2 * h:], e1_W2, e1_b2[None, :],
        p_W1, p_b1[None, :], p_W2[:, 0][None, :], p_b2[None, :])
    return out[:, 0]


# speedup vs baseline: 1.7901x; 1.7901x over previous
import functools

import jax
import jax.numpy as jnp
from jax import lax
from jax.experimental import pallas as pl
from jax.experimental.pallas import tpu as pltpu
from jax.experimental.pallas import tpu_sc as plsc

f32 = jnp.float32

_NC = 2
_NS = 16
_NW = _NC * _NS
_CH = 80

_EBLK = 2000
_NBLK = 2000


def _relu(v):
    return jnp.maximum(v, 0.0)


def _node_proj_body(x_ref, wa_ref, wb_ref, b_ref, a_ref, b_out_ref):
    xv = x_ref[...]
    a_ref[...] = jnp.dot(xv, wa_ref[...], preferred_element_type=f32)
    b_out_ref[...] = jnp.dot(xv, wb_ref[...], preferred_element_type=f32) + b_ref[...]


def _tc_node_proj(x, wa, wb, b1):
    n, d = x.shape
    h = wa.shape[1]
    blk = _NBLK
    return pl.pallas_call(
        _node_proj_body,
        grid=(n // blk,),
        in_specs=[
            pl.BlockSpec((blk, d), lambda i: (i, 0)),
            pl.BlockSpec((d, h), lambda i: (0, 0)),
            pl.BlockSpec((d, h), lambda i: (0, 0)),
            pl.BlockSpec((1, h), lambda i: (0, 0)),
        ],
        out_specs=[
            pl.BlockSpec((blk, h), lambda i: (i, 0)),
            pl.BlockSpec((blk, h), lambda i: (i, 0)),
        ],
        out_shape=[
            jax.ShapeDtypeStruct((n, h), f32),
            jax.ShapeDtypeStruct((n, h), f32),
        ],
    )(x, wa, wb, b1)


def _edge0_body(gs_ref, gd_ref, ea_ref, wc_ref, w2_ref, b2_ref, out_ref):
    pre = gs_ref[...] + gd_ref[...] + jnp.dot(
        ea_ref[...], wc_ref[...], preferred_element_type=f32)
    hv = _relu(pre)
    out_ref[...] = jnp.dot(hv, w2_ref[...], preferred_element_type=f32) + b2_ref[...]


def _tc_edge0(gs, gd, ea, wc, w2, b2):
    e, h = gs.shape
    de = ea.shape[1]
    blk = _EBLK
    return pl.pallas_call(
        _edge0_body,
        grid=(e // blk,),
        in_specs=[
            pl.BlockSpec((blk, h), lambda i: (i, 0)),
            pl.BlockSpec((blk, h), lambda i: (i, 0)),
            pl.BlockSpec((blk, de), lambda i: (i, 0)),
            pl.BlockSpec((de, h), lambda i: (0, 0)),
            pl.BlockSpec((h, h), lambda i: (0, 0)),
            pl.BlockSpec((1, h), lambda i: (0, 0)),
        ],
        out_specs=pl.BlockSpec((blk, h), lambda i: (i, 0)),
        out_shape=jax.ShapeDtypeStruct((e, h), f32),
    )(gs, gd, ea, wc, w2, b2)


def _node_body(x_ref, sd_ref, cnt_ref, ss_ref, wx_ref, wp_ref, wf_ref, b1_ref,
               w2_ref, b2_ref, wa_ref, wb_ref, eb1_ref, a_out, b_out):
    inv = 1.0 / jnp.maximum(cnt_ref[..., 0:1], 1.0)
    past = sd_ref[...] * inv
    xv = x_ref[...]
    hv = _relu(
        jnp.dot(xv, wx_ref[...], preferred_element_type=f32)
        + jnp.dot(past, wp_ref[...], preferred_element_type=f32)
        + jnp.dot(ss_ref[...], wf_ref[...], preferred_element_type=f32)
        + b1_ref[...])
    x0 = jnp.dot(hv, w2_ref[...], preferred_element_type=f32) + b2_ref[...] + xv
    a_out[...] = jnp.dot(x0, wa_ref[...], preferred_element_type=f32)
    b_out[...] = jnp.dot(x0, wb_ref[...], preferred_element_type=f32) + eb1_ref[...]


def _tc_node(x, sd, cnt, ss, wx, wp, wf, b1, w2, b2, wa, wb, eb1):
    n, d = x.shape
    h = w2.shape[1]
    blk = _NBLK
    wspec = pl.BlockSpec((d, h), lambda i: (0, 0))
    bspec = pl.BlockSpec((1, h), lambda i: (0, 0))
    return pl.pallas_call(
        _node_body,
        grid=(n // blk,),
        in_specs=[
            pl.BlockSpec((blk, d), lambda i: (i, 0)),
            pl.BlockSpec((blk, h), lambda i: (i, 0)),
            pl.BlockSpec((blk, cnt.shape[1]), lambda i: (i, 0)),
            pl.BlockSpec((blk, h), lambda i: (i, 0)),
            wspec, wspec, wspec, bspec, wspec, bspec, wspec, wspec, bspec,
        ],
        out_specs=[
            pl.BlockSpec((blk, h), lambda i: (i, 0)),
            pl.BlockSpec((blk, h), lambda i: (i, 0)),
        ],
        out_shape=[
            jax.ShapeDtypeStruct((n, h), f32),
            jax.ShapeDtypeStruct((n, h), f32),
        ],
    )(x, sd, cnt, ss, wx, wp, wf, b1, w2, b2, wa, wb, eb1)


def _edge1_body(gs_ref, gd_ref, ea_ref, wc_ref, w2_ref, b2_ref,
                pw1_ref, pb1_ref, pw2_ref, pb2_ref, out_ref):
    ea = ea_ref[...]
    hv = _relu(gs_ref[...] + gd_ref[...] + jnp.dot(
        ea, wc_ref[...], preferred_element_type=f32))
    ea1 = jnp.dot(hv, w2_ref[...], preferred_element_type=f32) + b2_ref[...] + ea
    ph = _relu(jnp.dot(ea1, pw1_ref[...], preferred_element_type=f32) + pb1_ref[...])
    out_ref[...] = jnp.sum(ph * pw2_ref[...], axis=1, keepdims=True) + pb2_ref[...]


def _tc_edge1(gs, gd, ea, wc, w2, b2, pw1, pb1, pw2row, pb2):
    e, h = gs.shape
    blk = _EBLK
    wspec = pl.BlockSpec((h, h), lambda i: (0, 0))
    bspec = pl.BlockSpec((1, h), lambda i: (0, 0))
    return pl.pallas_call(
        _edge1_body,
        grid=(e // blk,),
        in_specs=[
            pl.BlockSpec((blk, h), lambda i: (i, 0)),
            pl.BlockSpec((blk, h), lambda i: (i, 0)),
            pl.BlockSpec((blk, h), lambda i: (i, 0)),
            wspec, wspec, bspec, wspec, bspec, bspec,
            pl.BlockSpec((1, 1), lambda i: (0, 0)),
        ],
        out_specs=pl.BlockSpec((blk, 1), lambda i: (i, 0)),
        out_shape=jax.ShapeDtypeStruct((e, 1), f32),
    )(gs, gd, ea, wc, w2, b2, pw1, pb1, pw2row, pb2)


def _sc_mesh(num_cores=_NC):
    return plsc.VectorSubcoreMesh(
        core_axis_name="c", subcore_axis_name="s",
        num_cores=num_cores, num_subcores=_NS)


def _sc_gather_body(ta, tb, srci, dsti, out_s, out_d,
                    sidx, didx, rows_a, rows_b, sem_a, sem_b):
    ew = srci.shape[0] // _NW
    nch = ew // _CH
    wid = lax.axis_index("s") * _NC + lax.axis_index("c")

    def chunk(i, carry):
        base = pl.multiple_of(wid * ew + i * _CH, _CH)
        pltpu.sync_copy(srci.at[pl.ds(base, _CH)], sidx)
        pltpu.sync_copy(dsti.at[pl.ds(base, _CH)], didx)
        ca = pltpu.async_copy(ta.at[sidx], rows_a, sem_a)
        cb = pltpu.async_copy(tb.at[didx], rows_b, sem_b)
        ca.wait()
        cb.wait()
        pltpu.sync_copy(rows_a, out_s.at[pl.ds(base, _CH)])
        pltpu.sync_copy(rows_b, out_d.at[pl.ds(base, _CH)])
        return carry

    lax.fori_loop(0, nch, chunk, 0)


def _sc_gather2(ta, tb, srci, dsti):
    e = srci.shape[0]
    d = ta.shape[1]
    return pl.kernel(
        _sc_gather_body,
        out_type=(
            jax.ShapeDtypeStruct((e, d), f32),
            jax.ShapeDtypeStruct((e, d), f32),
        ),
        mesh=_sc_mesh(),
        scratch_types=[
            pltpu.VMEM((_CH,), jnp.int32),
            pltpu.VMEM((_CH,), jnp.int32),
            pltpu.VMEM((_CH, d), f32),
            pltpu.VMEM((_CH, d), f32),
            pltpu.SemaphoreType.DMA,
            pltpu.SemaphoreType.DMA,
        ],
    )(ta, tb, srci, dsti)


def _sc_scatter_body(ea, cat_idx, zacc, sums, cnts,
                     idx_v, idx2_v, rows_v, acc):
    nacc = zacc.shape[0]
    rng = nacc - 512
    e = ea.shape[0]
    ept = e // _NS
    nch = ept // _CH
    rpt = nacc // _NS
    c = lax.axis_index("c")
    s = lax.axis_index("s")
    r0 = pl.multiple_of(s * rpt, 8)

    o16v = jnp.full((16,), 1.0, f32)

    def _phase(p, out_ref, with_rows):
        pltpu.sync_copy(zacc.at[pl.ds(r0, rpt)], acc.at[pl.ds(r0, rpt)])
        if not with_rows:
            def orow(j, carry):
                for k in range(8):
                    rows_v[j, k * 16:(k + 1) * 16] = o16v
                return carry

            lax.fori_loop(0, _CH, orow, 0)
        plsc.subcore_barrier()
        lo = p * rng

        def chunk(i, carry):
            ibase = pl.multiple_of(c * e + s * ept + i * _CH, _CH)
            pltpu.sync_copy(cat_idx.at[pl.ds(ibase, _CH)], idx_v)
            if with_rows:
                ebase = pl.multiple_of(s * ept + i * _CH, _CH)
                pltpu.sync_copy(ea.at[pl.ds(ebase, _CH)], rows_v)
            for k in range(_CH // 16):
                v = idx_v[pl.ds(k * 16, 16)]
                local = v - lo
                ok = (local >= 0) & (local < rng)
                idx2_v[pl.ds(k * 16, 16)] = jnp.where(ok, local, rng)
            pltpu.sync_copy(rows_v, acc.at[idx2_v], add=True)
            return carry

        lax.fori_loop(0, nch, chunk, 0)
        plsc.subcore_barrier()
        w0 = pl.multiple_of((c * 2 + p) * nacc + r0, 8)
        pltpu.sync_copy(acc.at[pl.ds(r0, rpt)], out_ref.at[pl.ds(w0, rpt)])

    for p in range(2):
        _phase(p, sums, True)
    for p in range(2):
        _phase(p, cnts, False)


def _sc_scatter(ea, cat_idx, zacc):
    nacc = zacc.shape[0]
    h = ea.shape[1]
    return pl.kernel(
        _sc_scatter_body,
        out_type=(
            jax.ShapeDtypeStruct((4 * nacc, h), f32),
            jax.ShapeDtypeStruct((4 * nacc, h), f32),
        ),
        mesh=_sc_mesh(),
        scratch_types=[
            pltpu.VMEM((_CH,), jnp.int32),
            pltpu.VMEM((_CH,), jnp.int32),
            pltpu.VMEM((_CH, h), f32),
            pltpu.VMEM_SHARED((nacc, h), f32),
        ],
    )(ea, cat_idx, zacc)


def kernel(x, edge_index, edge_attr,
           e0_W1, e0_b1, e0_W2, e0_b2, n0_W1, n0_b1, n0_W2, n0_b2,
           e1_W1, e1_b1, e1_W2, e1_b2, n1_W1, n1_b1, n1_W2, n1_b2,
           p_W1, p_b1, p_W2, p_b2):
    n, d = x.shape
    h = e0_W2.shape[1]
    src = edge_index[0]
    dst = edge_index[1]

    xa, xb = _tc_node_proj(x, e0_W1[:d], e0_W1[d:2 * d], e0_b1[None, :])
    gs, gd = _sc_gather2(xa, xb, src, dst)
    ea0 = _tc_edge0(gs, gd, edge_attr, e0_W1[2 * d:], e0_W2, e0_b2[None, :])

    npad = ((n + 1023) // 1024) * 1024
    rng = npad // 2
    nacc = rng + 512
    zacc = jnp.zeros((nacc, h), f32)
    cat_idx = jnp.concatenate([dst, src])
    sums4, cnts4 = _sc_scatter(ea0, cat_idx, zacc)
    sd = jnp.concatenate([sums4[:rng], sums4[nacc:nacc + (n - rng)]])
    ss = jnp.concatenate([sums4[2 * nacc:2 * nacc + rng],
                          sums4[3 * nacc:3 * nacc + (n - rng)]])
    cnt = jnp.concatenate([cnts4[:rng], cnts4[nacc:nacc + (n - rng)]])

    a1, b1v = _tc_node(
        x, sd, cnt, ss,
        n0_W1[:d], n0_W1[d:d + h], n0_W1[d + h:], n0_b1[None, :],
        n0_W2, n0_b2[None, :],
        e1_W1[:h], e1_W1[h:2 * h], e1_b1[None, :])

    g1s, g1d = _sc_gather2(a1, b1v, src, dst)

    out = _tc_edge1(
        g1s, g1d, ea0, e1_W1[2 * h:], e1_W2, e1_b2[None, :],
        p_W1, p_b1[None, :], p_W2[:, 0][None, :], p_b2[None, :])
    return out[:, 0]


# speedup vs baseline: 2.3265x; 1.2996x over previous
import functools

import jax
import jax.numpy as jnp
from jax import lax
from jax.experimental import pallas as pl
from jax.experimental.pallas import tpu as pltpu
from jax.experimental.pallas import tpu_sc as plsc

f32 = jnp.float32

_NC = 2
_NS = 16
_NW = _NC * _NS
_CH = 80
_CHG = 400

_EBLK = 2000
_NBLK = 2000


def _relu(v):
    return jnp.maximum(v, 0.0)


def _node_proj_body(x_ref, wa_ref, wb_ref, b_ref, a_ref, b_out_ref):
    xv = x_ref[...]
    a_ref[...] = jnp.dot(xv, wa_ref[...], preferred_element_type=f32)
    b_out_ref[...] = jnp.dot(xv, wb_ref[...], preferred_element_type=f32) + b_ref[...]


def _tc_node_proj(x, wa, wb, b1):
    n, d = x.shape
    h = wa.shape[1]
    blk = _NBLK
    return pl.pallas_call(
        _node_proj_body,
        grid=(n // blk,),
        in_specs=[
            pl.BlockSpec((blk, d), lambda i: (i, 0)),
            pl.BlockSpec((d, h), lambda i: (0, 0)),
            pl.BlockSpec((d, h), lambda i: (0, 0)),
            pl.BlockSpec((1, h), lambda i: (0, 0)),
        ],
        out_specs=[
            pl.BlockSpec((blk, h), lambda i: (i, 0)),
            pl.BlockSpec((blk, h), lambda i: (i, 0)),
        ],
        out_shape=[
            jax.ShapeDtypeStruct((n, h), f32),
            jax.ShapeDtypeStruct((n, h), f32),
        ],
    )(x, wa, wb, b1)


def _edge0_body(gs_ref, gd_ref, ea_ref, wc_ref, w2_ref, b2_ref, out_ref):
    pre = gs_ref[...] + gd_ref[...] + jnp.dot(
        ea_ref[...], wc_ref[...], preferred_element_type=f32)
    hv = _relu(pre)
    out_ref[...] = jnp.dot(hv, w2_ref[...], preferred_element_type=f32) + b2_ref[...]


def _tc_edge0(gs, gd, ea, wc, w2, b2):
    e, h = gs.shape
    de = ea.shape[1]
    blk = _EBLK
    return pl.pallas_call(
        _edge0_body,
        grid=(e // blk,),
        in_specs=[
            pl.BlockSpec((blk, h), lambda i: (i, 0)),
            pl.BlockSpec((blk, h), lambda i: (i, 0)),
            pl.BlockSpec((blk, de), lambda i: (i, 0)),
            pl.BlockSpec((de, h), lambda i: (0, 0)),
            pl.BlockSpec((h, h), lambda i: (0, 0)),
            pl.BlockSpec((1, h), lambda i: (0, 0)),
        ],
        out_specs=pl.BlockSpec((blk, h), lambda i: (i, 0)),
        out_shape=jax.ShapeDtypeStruct((e, h), f32),
    )(gs, gd, ea, wc, w2, b2)


def _node_body(x_ref, sd_ref, cnt_ref, ss_ref, wx_ref, wp_ref, wf_ref, b1_ref,
               w2_ref, b2_ref, wa_ref, wb_ref, eb1_ref, a_out, b_out):
    inv = 1.0 / jnp.maximum(cnt_ref[..., 0:1], 1.0)
    past = sd_ref[...] * inv
    xv = x_ref[...]
    hv = _relu(
        jnp.dot(xv, wx_ref[...], preferred_element_type=f32)
        + jnp.dot(past, wp_ref[...], preferred_element_type=f32)
        + jnp.dot(ss_ref[...], wf_ref[...], preferred_element_type=f32)
        + b1_ref[...])
    x0 = jnp.dot(hv, w2_ref[...], preferred_element_type=f32) + b2_ref[...] + xv
    a_out[...] = jnp.dot(x0, wa_ref[...], preferred_element_type=f32)
    b_out[...] = jnp.dot(x0, wb_ref[...], preferred_element_type=f32) + eb1_ref[...]


def _tc_node(x, sd, cnt, ss, wx, wp, wf, b1, w2, b2, wa, wb, eb1):
    n, d = x.shape
    h = w2.shape[1]
    blk = _NBLK
    wspec = pl.BlockSpec((d, h), lambda i: (0, 0))
    bspec = pl.BlockSpec((1, h), lambda i: (0, 0))
    return pl.pallas_call(
        _node_body,
        grid=(n // blk,),
        in_specs=[
            pl.BlockSpec((blk, d), lambda i: (i, 0)),
            pl.BlockSpec((blk, h), lambda i: (i, 0)),
            pl.BlockSpec((blk, cnt.shape[1]), lambda i: (i, 0)),
            pl.BlockSpec((blk, h), lambda i: (i, 0)),
            wspec, wspec, wspec, bspec, wspec, bspec, wspec, wspec, bspec,
        ],
        out_specs=[
            pl.BlockSpec((blk, h), lambda i: (i, 0)),
            pl.BlockSpec((blk, h), lambda i: (i, 0)),
        ],
        out_shape=[
            jax.ShapeDtypeStruct((n, h), f32),
            jax.ShapeDtypeStruct((n, h), f32),
        ],
    )(x, sd, cnt, ss, wx, wp, wf, b1, w2, b2, wa, wb, eb1)


def _edge1_body(gs_ref, gd_ref, ea_ref, wc_ref, w2_ref, b2_ref,
                pw1_ref, pb1_ref, pw2_ref, pb2_ref, out_ref):
    ea = ea_ref[...]
    hv = _relu(gs_ref[...] + gd_ref[...] + jnp.dot(
        ea, wc_ref[...], preferred_element_type=f32))
    ea1 = jnp.dot(hv, w2_ref[...], preferred_element_type=f32) + b2_ref[...] + ea
    ph = _relu(jnp.dot(ea1, pw1_ref[...], preferred_element_type=f32) + pb1_ref[...])
    out_ref[...] = jnp.sum(ph * pw2_ref[...], axis=1, keepdims=True) + pb2_ref[...]


def _tc_edge1(gs, gd, ea, wc, w2, b2, pw1, pb1, pw2row, pb2):
    e, h = gs.shape
    blk = _EBLK
    wspec = pl.BlockSpec((h, h), lambda i: (0, 0))
    bspec = pl.BlockSpec((1, h), lambda i: (0, 0))
    return pl.pallas_call(
        _edge1_body,
        grid=(e // blk,),
        in_specs=[
            pl.BlockSpec((blk, h), lambda i: (i, 0)),
            pl.BlockSpec((blk, h), lambda i: (i, 0)),
            pl.BlockSpec((blk, h), lambda i: (i, 0)),
            wspec, wspec, bspec, wspec, bspec, bspec,
            pl.BlockSpec((1, 1), lambda i: (0, 0)),
        ],
        out_specs=pl.BlockSpec((blk, 1), lambda i: (i, 0)),
        out_shape=jax.ShapeDtypeStruct((e, 1), f32),
    )(gs, gd, ea, wc, w2, b2, pw1, pb1, pw2row, pb2)


def _sc_mesh(num_cores=_NC):
    return plsc.VectorSubcoreMesh(
        core_axis_name="c", subcore_axis_name="s",
        num_cores=num_cores, num_subcores=_NS)


def _sc_gather_body(ta, tb, srci, dsti, out_s, out_d,
                    sidx, didx, rows_a, rows_b, sem_a, sem_b):
    ew = srci.shape[0] // _NW
    nch = ew // _CHG
    nsub = _CHG // _CH
    wid = lax.axis_index("s") * _NC + lax.axis_index("c")

    def chunk(i, carry):
        base = pl.multiple_of(wid * ew + i * _CHG, _CH)
        ia = pltpu.async_copy(srci.at[pl.ds(base, _CHG)], sidx, sem_a)
        ib = pltpu.async_copy(dsti.at[pl.ds(base, _CHG)], didx, sem_b)
        ia.wait()
        ib.wait()
        ga = []
        gb = []
        for k in range(nsub):
            sl = pl.ds(k * _CH, _CH)
            ga.append(pltpu.async_copy(ta.at[sidx.at[sl]], rows_a.at[sl], sem_a))
            gb.append(pltpu.async_copy(tb.at[didx.at[sl]], rows_b.at[sl], sem_b))
        for op in ga + gb:
            op.wait()
        pltpu.sync_copy(rows_a, out_s.at[pl.ds(base, _CHG)])
        pltpu.sync_copy(rows_b, out_d.at[pl.ds(base, _CHG)])
        return carry

    lax.fori_loop(0, nch, chunk, 0)


def _sc_gather2(ta, tb, srci, dsti):
    e = srci.shape[0]
    d = ta.shape[1]
    return pl.kernel(
        _sc_gather_body,
        out_type=(
            jax.ShapeDtypeStruct((e, d), f32),
            jax.ShapeDtypeStruct((e, d), f32),
        ),
        mesh=_sc_mesh(),
        scratch_types=[
            pltpu.VMEM((_CHG,), jnp.int32),
            pltpu.VMEM((_CHG,), jnp.int32),
            pltpu.VMEM((_CHG, d), f32),
            pltpu.VMEM((_CHG, d), f32),
            pltpu.SemaphoreType.DMA,
            pltpu.SemaphoreType.DMA,
        ],
    )(ta, tb, srci, dsti)


def _sc_scatter_body(ea, cat_idx, zacc, sums, cnts,
                     idx_v, i2a, i2b, i2c, i2d, i2e, rows_v, acc, sem):
    nacc = zacc.shape[0]
    rng = nacc - 512
    e = ea.shape[0]
    ept = e // _NS
    nch = ept // _CHG
    nsub = _CHG // _CH
    rpt = nacc // _NS
    c = lax.axis_index("c")
    s = lax.axis_index("s")
    r0 = pl.multiple_of(s * rpt, 8)
    idx2 = [i2a, i2b, i2c, i2d, i2e]

    o16v = jnp.full((16,), 1.0, f32)

    def _phase(p, out_ref, with_rows):
        pltpu.sync_copy(zacc.at[pl.ds(r0, rpt)], acc.at[pl.ds(r0, rpt)])
        if not with_rows:
            def orow(j, carry):
                for k in range(8):
                    rows_v[j, k * 16:(k + 1) * 16] = o16v
                return carry

            lax.fori_loop(0, _CHG, orow, 0)
        plsc.subcore_barrier()
        lo = p * rng

        def chunk(i, carry):
            ibase = pl.multiple_of(c * e + s * ept + i * _CHG, _CH)
            pltpu.sync_copy(cat_idx.at[pl.ds(ibase, _CHG)], idx_v)
            if with_rows:
                ebase = pl.multiple_of(s * ept + i * _CHG, _CH)
                pltpu.sync_copy(ea.at[pl.ds(ebase, _CHG)], rows_v)
            for k in range(_CHG // 16):
                v = idx_v[pl.ds(k * 16, 16)]
                local = v - lo
                ok = (local >= 0) & (local < rng)
                idx2[k // 5][pl.ds((k % 5) * 16, 16)] = jnp.where(ok, local, rng)
            ops = []
            for k in range(nsub):
                ops.append(pltpu.async_copy(
                    rows_v.at[pl.ds(k * _CH, _CH)], acc.at[idx2[k]], sem,
                    add=True))
            for op in ops:
                op.wait()
            return carry

        lax.fori_loop(0, nch, chunk, 0)
        plsc.subcore_barrier()
        w0 = pl.multiple_of((c * 2 + p) * nacc + r0, 8)
        pltpu.sync_copy(acc.at[pl.ds(r0, rpt)], out_ref.at[pl.ds(w0, rpt)])

    for p in range(2):
        _phase(p, sums, True)
    for p in range(2):
        _phase(p, cnts, False)


def _sc_scatter(ea, cat_idx, zacc):
    nacc = zacc.shape[0]
    h = ea.shape[1]
    return pl.kernel(
        _sc_scatter_body,
        out_type=(
            jax.ShapeDtypeStruct((4 * nacc, h), f32),
            jax.ShapeDtypeStruct((4 * nacc, h), f32),
        ),
        mesh=_sc_mesh(),
        scratch_types=[
            pltpu.VMEM((_CHG,), jnp.int32),
            pltpu.VMEM((_CH,), jnp.int32),
            pltpu.VMEM((_CH,), jnp.int32),
            pltpu.VMEM((_CH,), jnp.int32),
            pltpu.VMEM((_CH,), jnp.int32),
            pltpu.VMEM((_CH,), jnp.int32),
            pltpu.VMEM((_CHG, h), f32),
            pltpu.VMEM_SHARED((nacc, h), f32),
            pltpu.SemaphoreType.DMA,
        ],
    )(ea, cat_idx, zacc)


def kernel(x, edge_index, edge_attr,
           e0_W1, e0_b1, e0_W2, e0_b2, n0_W1, n0_b1, n0_W2, n0_b2,
           e1_W1, e1_b1, e1_W2, e1_b2, n1_W1, n1_b1, n1_W2, n1_b2,
           p_W1, p_b1, p_W2, p_b2):
    n, d = x.shape
    h = e0_W2.shape[1]
    src = edge_index[0]
    dst = edge_index[1]

    xa, xb = _tc_node_proj(x, e0_W1[:d], e0_W1[d:2 * d], e0_b1[None, :])
    gs, gd = _sc_gather2(xa, xb, src, dst)
    ea0 = _tc_edge0(gs, gd, edge_attr, e0_W1[2 * d:], e0_W2, e0_b2[None, :])

    npad = ((n + 1023) // 1024) * 1024
    rng = npad // 2
    nacc = rng + 512
    zacc = jnp.zeros((nacc, h), f32)
    cat_idx = jnp.concatenate([dst, src])
    sums4, cnts4 = _sc_scatter(ea0, cat_idx, zacc)
    sd = jnp.concatenate([sums4[:rng], sums4[nacc:nacc + (n - rng)]])
    ss = jnp.concatenate([sums4[2 * nacc:2 * nacc + rng],
                          sums4[3 * nacc:3 * nacc + (n - rng)]])
    cnt = jnp.concatenate([cnts4[:rng], cnts4[nacc:nacc + (n - rng)]])

    a1, b1v = _tc_node(
        x, sd, cnt, ss,
        n0_W1[:d], n0_W1[d:d + h], n0_W1[d + h:], n0_b1[None, :],
        n0_W2, n0_b2[None, :],
        e1_W1[:h], e1_W1[h:2 * h], e1_b1[None, :])

    g1s, g1d = _sc_gather2(a1, b1v, src, dst)

    out = _tc_edge1(
        g1s, g1d, ea0, e1_W1[2 * h:], e1_W2, e1_b2[None, :],
        p_W1, p_b1[None, :], p_W2[:, 0][None, :], p_b2[None, :])
    return out[:, 0]


# speedup vs baseline: 2.5585x; 1.0997x over previous
import functools

import jax
import jax.numpy as jnp
from jax import lax
from jax.experimental import pallas as pl
from jax.experimental.pallas import tpu as pltpu
from jax.experimental.pallas import tpu_sc as plsc

f32 = jnp.float32

_NC = 2
_NS = 16
_NW = _NC * _NS
_CH = 80
_CHG = 400

_EBLK = 2000
_NBLK = 2000


def _relu(v):
    return jnp.maximum(v, 0.0)


def _node_proj_body(x_ref, wa_ref, wb_ref, b_ref, a_ref, b_out_ref):
    xv = x_ref[...]
    a_ref[...] = jnp.dot(xv, wa_ref[...], preferred_element_type=f32)
    b_out_ref[...] = jnp.dot(xv, wb_ref[...], preferred_element_type=f32) + b_ref[...]


def _tc_node_proj(x, wa, wb, b1):
    n, d = x.shape
    h = wa.shape[1]
    blk = _NBLK
    return pl.pallas_call(
        _node_proj_body,
        grid=(n // blk,),
        in_specs=[
            pl.BlockSpec((blk, d), lambda i: (i, 0)),
            pl.BlockSpec((d, h), lambda i: (0, 0)),
            pl.BlockSpec((d, h), lambda i: (0, 0)),
            pl.BlockSpec((1, h), lambda i: (0, 0)),
        ],
        out_specs=[
            pl.BlockSpec((blk, h), lambda i: (i, 0)),
            pl.BlockSpec((blk, h), lambda i: (i, 0)),
        ],
        out_shape=[
            jax.ShapeDtypeStruct((n, h), f32),
            jax.ShapeDtypeStruct((n, h), f32),
        ],
    )(x, wa, wb, b1)


def _edge0_body(gs_ref, gd_ref, ea_ref, wc_ref, w2_ref, b2_ref, out_ref):
    pre = gs_ref[...] + gd_ref[...] + jnp.dot(
        ea_ref[...], wc_ref[...], preferred_element_type=f32)
    hv = _relu(pre)
    out_ref[...] = jnp.dot(hv, w2_ref[...], preferred_element_type=f32) + b2_ref[...]


def _tc_edge0(gs, gd, ea, wc, w2, b2):
    e, h = gs.shape
    de = ea.shape[1]
    blk = _EBLK
    return pl.pallas_call(
        _edge0_body,
        grid=(e // blk,),
        in_specs=[
            pl.BlockSpec((blk, h), lambda i: (i, 0)),
            pl.BlockSpec((blk, h), lambda i: (i, 0)),
            pl.BlockSpec((blk, de), lambda i: (i, 0)),
            pl.BlockSpec((de, h), lambda i: (0, 0)),
            pl.BlockSpec((h, h), lambda i: (0, 0)),
            pl.BlockSpec((1, h), lambda i: (0, 0)),
        ],
        out_specs=pl.BlockSpec((blk, h), lambda i: (i, 0)),
        out_shape=jax.ShapeDtypeStruct((e, h), f32),
    )(gs, gd, ea, wc, w2, b2)


def _node_body(x_ref, sd_ref, cnt_ref, ss_ref, wx_ref, wp_ref, wf_ref, b1_ref,
               w2_ref, b2_ref, wa_ref, wb_ref, eb1_ref, a_out, b_out):
    inv = 1.0 / jnp.maximum(cnt_ref[..., 0:1], 1.0)
    past = sd_ref[...] * inv
    xv = x_ref[...]
    hv = _relu(
        jnp.dot(xv, wx_ref[...], preferred_element_type=f32)
        + jnp.dot(past, wp_ref[...], preferred_element_type=f32)
        + jnp.dot(ss_ref[...], wf_ref[...], preferred_element_type=f32)
        + b1_ref[...])
    x0 = jnp.dot(hv, w2_ref[...], preferred_element_type=f32) + b2_ref[...] + xv
    a_out[...] = jnp.dot(x0, wa_ref[...], preferred_element_type=f32)
    b_out[...] = jnp.dot(x0, wb_ref[...], preferred_element_type=f32) + eb1_ref[...]


def _tc_node(x, sd, cnt, ss, wx, wp, wf, b1, w2, b2, wa, wb, eb1):
    n, d = x.shape
    h = w2.shape[1]
    blk = _NBLK
    wspec = pl.BlockSpec((d, h), lambda i: (0, 0))
    bspec = pl.BlockSpec((1, h), lambda i: (0, 0))
    return pl.pallas_call(
        _node_body,
        grid=(n // blk,),
        in_specs=[
            pl.BlockSpec((blk, d), lambda i: (i, 0)),
            pl.BlockSpec((blk, h), lambda i: (i, 0)),
            pl.BlockSpec((blk, cnt.shape[1]), lambda i: (i, 0)),
            pl.BlockSpec((blk, h), lambda i: (i, 0)),
            wspec, wspec, wspec, bspec, wspec, bspec, wspec, wspec, bspec,
        ],
        out_specs=[
            pl.BlockSpec((blk, h), lambda i: (i, 0)),
            pl.BlockSpec((blk, h), lambda i: (i, 0)),
        ],
        out_shape=[
            jax.ShapeDtypeStruct((n, h), f32),
            jax.ShapeDtypeStruct((n, h), f32),
        ],
    )(x, sd, cnt, ss, wx, wp, wf, b1, w2, b2, wa, wb, eb1)


def _edge1_body(gs_ref, gd_ref, ea_ref, wc_ref, w2_ref, b2_ref,
                pw1_ref, pb1_ref, pw2_ref, pb2_ref, out_ref):
    ea = ea_ref[...]
    hv = _relu(gs_ref[...] + gd_ref[...] + jnp.dot(
        ea, wc_ref[...], preferred_element_type=f32))
    ea1 = jnp.dot(hv, w2_ref[...], preferred_element_type=f32) + b2_ref[...] + ea
    ph = _relu(jnp.dot(ea1, pw1_ref[...], preferred_element_type=f32) + pb1_ref[...])
    out_ref[...] = jnp.sum(ph * pw2_ref[...], axis=1, keepdims=True) + pb2_ref[...]


def _tc_edge1(gs, gd, ea, wc, w2, b2, pw1, pb1, pw2row, pb2):
    e, h = gs.shape
    blk = _EBLK
    wspec = pl.BlockSpec((h, h), lambda i: (0, 0))
    bspec = pl.BlockSpec((1, h), lambda i: (0, 0))
    return pl.pallas_call(
        _edge1_body,
        grid=(e // blk,),
        in_specs=[
            pl.BlockSpec((blk, h), lambda i: (i, 0)),
            pl.BlockSpec((blk, h), lambda i: (i, 0)),
            pl.BlockSpec((blk, h), lambda i: (i, 0)),
            wspec, wspec, bspec, wspec, bspec, bspec,
            pl.BlockSpec((1, 1), lambda i: (0, 0)),
        ],
        out_specs=pl.BlockSpec((blk, 1), lambda i: (i, 0)),
        out_shape=jax.ShapeDtypeStruct((e, 1), f32),
    )(gs, gd, ea, wc, w2, b2, pw1, pb1, pw2row, pb2)


def _sc_mesh(num_cores=_NC):
    return plsc.VectorSubcoreMesh(
        core_axis_name="c", subcore_axis_name="s",
        num_cores=num_cores, num_subcores=_NS)


def _sc_gather_body(ta, tb, srci, dsti, out_s, out_d,
                    sidx, didx, rows_a, rows_b, sem_a, sem_b):
    ew = srci.shape[0] // _NW
    nch = ew // _CHG
    nsub = _CHG // _CH
    wid = lax.axis_index("s") * _NC + lax.axis_index("c")

    def chunk(i, carry):
        base = pl.multiple_of(wid * ew + i * _CHG, _CH)
        ia = pltpu.async_copy(srci.at[pl.ds(base, _CHG)], sidx, sem_a)
        ib = pltpu.async_copy(dsti.at[pl.ds(base, _CHG)], didx, sem_b)
        ia.wait()
        ib.wait()
        ga = []
        gb = []
        for k in range(nsub):
            sl = pl.ds(k * _CH, _CH)
            ga.append(pltpu.async_copy(ta.at[sidx.at[sl]], rows_a.at[sl], sem_a))
            gb.append(pltpu.async_copy(tb.at[didx.at[sl]], rows_b.at[sl], sem_b))
        for op in ga + gb:
            op.wait()
        pltpu.sync_copy(rows_a, out_s.at[pl.ds(base, _CHG)])
        pltpu.sync_copy(rows_b, out_d.at[pl.ds(base, _CHG)])
        return carry

    lax.fori_loop(0, nch, chunk, 0)


def _sc_gather2(ta, tb, srci, dsti):
    e = srci.shape[0]
    d = ta.shape[1]
    return pl.kernel(
        _sc_gather_body,
        out_type=(
            jax.ShapeDtypeStruct((e, d), f32),
            jax.ShapeDtypeStruct((e, d), f32),
        ),
        mesh=_sc_mesh(),
        scratch_types=[
            pltpu.VMEM((_CHG,), jnp.int32),
            pltpu.VMEM((_CHG,), jnp.int32),
            pltpu.VMEM((_CHG, d), f32),
            pltpu.VMEM((_CHG, d), f32),
            pltpu.SemaphoreType.DMA,
            pltpu.SemaphoreType.DMA,
        ],
    )(ta, tb, srci, dsti)


def _sc_scatter_body(ea, cat_idx, zacc, sums, cnts,
                     idx_v, i2a, i2b, i2c, i2d, i2e, rows_v, acc, sem):
    nacc = zacc.shape[0]
    rng = nacc - 512
    e = ea.shape[0]
    ept = e // _NS
    nch = ept // _CHG
    nsub = _CHG // _CH
    rpt = nacc // _NS
    c = lax.axis_index("c")
    s = lax.axis_index("s")
    r0 = pl.multiple_of(s * rpt, 8)
    idx2 = [i2a, i2b, i2c, i2d, i2e]

    o16v = jnp.full((16,), 1.0, f32)

    def _phase(lo, ioff, w0, out_ref, with_rows):
        pltpu.sync_copy(zacc.at[pl.ds(r0, rpt)], acc.at[pl.ds(r0, rpt)])
        if not with_rows:
            def orow(j, carry):
                for k in range(8):
                    rows_v[j, k * 16:(k + 1) * 16] = o16v
                return carry

            lax.fori_loop(0, _CHG, orow, 0)
        plsc.subcore_barrier()

        def chunk(i, carry):
            ibase = pl.multiple_of(ioff + s * ept + i * _CHG, _CH)
            pltpu.sync_copy(cat_idx.at[pl.ds(ibase, _CHG)], idx_v)
            if with_rows:
                ebase = pl.multiple_of(s * ept + i * _CHG, _CH)
                pltpu.sync_copy(ea.at[pl.ds(ebase, _CHG)], rows_v)
            for k in range(_CHG // 16):
                v = idx_v[pl.ds(k * 16, 16)]
                local = v - lo
                ok = (local >= 0) & (local < rng)
                idx2[k // 5][pl.ds((k % 5) * 16, 16)] = jnp.where(ok, local, rng)
            ops = []
            for k in range(nsub):
                ops.append(pltpu.async_copy(
                    rows_v.at[pl.ds(k * _CH, _CH)], acc.at[idx2[k]], sem,
                    add=True))
            for op in ops:
                op.wait()
            return carry

        lax.fori_loop(0, nch, chunk, 0)
        plsc.subcore_barrier()
        pltpu.sync_copy(acc.at[pl.ds(r0, rpt)],
                        out_ref.at[pl.ds(pl.multiple_of(w0 + r0, 8), rpt)])

    for p in range(2):
        _phase(p * rng, c * e, (c * 2 + p) * nacc, sums, True)
    _phase(c * rng, 0, c * nacc, cnts, False)


def _sc_scatter(ea, cat_idx, zacc):
    nacc = zacc.shape[0]
    h = ea.shape[1]
    return pl.kernel(
        _sc_scatter_body,
        out_type=(
            jax.ShapeDtypeStruct((4 * nacc, h), f32),
            jax.ShapeDtypeStruct((2 * nacc, h), f32),
        ),
        mesh=_sc_mesh(),
        scratch_types=[
            pltpu.VMEM((_CHG,), jnp.int32),
            pltpu.VMEM((_CH,), jnp.int32),
            pltpu.VMEM((_CH,), jnp.int32),
            pltpu.VMEM((_CH,), jnp.int32),
            pltpu.VMEM((_CH,), jnp.int32),
            pltpu.VMEM((_CH,), jnp.int32),
            pltpu.VMEM((_CHG, h), f32),
            pltpu.VMEM_SHARED((nacc, h), f32),
            pltpu.SemaphoreType.DMA,
        ],
    )(ea, cat_idx, zacc)


def kernel(x, edge_index, edge_attr,
           e0_W1, e0_b1, e0_W2, e0_b2, n0_W1, n0_b1, n0_W2, n0_b2,
           e1_W1, e1_b1, e1_W2, e1_b2, n1_W1, n1_b1, n1_W2, n1_b2,
           p_W1, p_b1, p_W2, p_b2):
    n, d = x.shape
    h = e0_W2.shape[1]
    src = edge_index[0]
    dst = edge_index[1]

    xa, xb = _tc_node_proj(x, e0_W1[:d], e0_W1[d:2 * d], e0_b1[None, :])
    gs, gd = _sc_gather2(xa, xb, src, dst)
    ea0 = _tc_edge0(gs, gd, edge_attr, e0_W1[2 * d:], e0_W2, e0_b2[None, :])

    npad = ((n + 1023) // 1024) * 1024
    rng = npad // 2
    nacc = rng + 512
    zacc = jnp.zeros((nacc, h), f32)
    cat_idx = jnp.concatenate([dst, src])
    sums4, cnts4 = _sc_scatter(ea0, cat_idx, zacc)
    sd = jnp.concatenate([sums4[:rng], sums4[nacc:nacc + (n - rng)]])
    ss = jnp.concatenate([sums4[2 * nacc:2 * nacc + rng],
                          sums4[3 * nacc:3 * nacc + (n - rng)]])
    cnt = jnp.concatenate([cnts4[:rng], cnts4[nacc:nacc + (n - rng)]])

    a1, b1v = _tc_node(
        x, sd, cnt, ss,
        n0_W1[:d], n0_W1[d:d + h], n0_W1[d + h:], n0_b1[None, :],
        n0_W2, n0_b2[None, :],
        e1_W1[:h], e1_W1[h:2 * h], e1_b1[None, :])

    g1s, g1d = _sc_gather2(a1, b1v, src, dst)

    out = _tc_edge1(
        g1s, g1d, ea0, e1_W1[2 * h:], e1_W2, e1_b2[None, :],
        p_W1, p_b1[None, :], p_W2[:, 0][None, :], p_b2[None, :])
    return out[:, 0]
